# Initial kernel scaffold; baseline (speedup 1.0000x reference)
import jax
import jax.numpy as jnp
from jax.experimental import pallas as pl


def kernel(x, om):
    raise NotImplementedError("write your pallas kernel here")


# speedup vs baseline: 26.0495x; 26.0495x over previous
import functools
import itertools

import jax
import jax.numpy as jnp
import numpy as np
from jax import lax
from jax.experimental import pallas as pl
from jax.experimental.pallas import tpu as pltpu
from jax.experimental.pallas import tpu_sc as plsc

IM_SIZE = (256, 256)
GRID_SIZE = (512, 512)
NUMPOINTS = (6, 6)
TABLE_OVERSAMP = (1024, 1024)
N_SHIFT = (128.0, 128.0)
ALPHA = (2.34 * 6, 2.34 * 6)
NCOIL = 8
KLEN = 131072

PAD = NUMPOINTS[0] - 1
PG = GRID_SIZE[0] + PAD
NROWS = PG * PG
TAPS = NUMPOINTS[0] * NUMPOINTS[1]

NW = 32
KW = KLEN // NW
CH = 64
NCH = KW // CH
NSEG = TAPS * CH // 128
TLEN = NUMPOINTS[0] * (TABLE_OVERSAMP[0] + 1) * 2


def _kb_kernel_np(u, J, alpha):
    u = np.asarray(u, dtype=np.float64)
    mask = np.abs(u) < (J / 2.0)
    vals = np.zeros_like(u)
    arg = np.sqrt(np.maximum(1.0 - (u[mask] / (J / 2.0)) ** 2, 0.0))
    vals[mask] = np.i0(alpha * arg) / np.i0(alpha)
    return vals


def _build_table_1d(J, L, K, N, alpha):
    m = np.arange(J * L + 1)
    u = -J / 2.0 + m / float(L)
    coef = _kb_kernel_np(u, J, alpha)
    gam = 2.0 * np.pi / K
    h = coef * np.exp(1j * gam * (N - 1) / 2.0 * u)
    h[-1] = 0.0
    return h.astype(np.complex64)


def _kb_ft_np(om, J, alpha):
    z = np.sqrt((2.0 * np.pi * (J / 2.0) * np.asarray(om, dtype=np.float64)) ** 2 - alpha ** 2 + 0j)
    return np.real(J * np.sinc(z / np.pi) / np.i0(alpha))


def _make_scaling():
    sc = None
    for N, K, J, alpha in zip(IM_SIZE, GRID_SIZE, NUMPOINTS, ALPHA):
        n = np.arange(N) - (N - 1) / 2.0
        s1 = 1.0 / _kb_ft_np(n / K, J, alpha)
        sc = s1 if sc is None else sc[:, None] * s1[None, :]
    return sc.astype(np.complex64)


def _pack_table(d):
    tab = _build_table_1d(NUMPOINTS[d], TABLE_OVERSAMP[d], GRID_SIZE[d], IM_SIZE[d], ALPHA[d])
    L = TABLE_OVERSAMP[d]
    J = NUMPOINTS[d]
    q = np.arange(L + 1)
    idx = q[None, :] + (J - 1) * L - L * np.arange(J)[:, None]
    t = tab[idx]
    return np.stack([t.real, t.imag], axis=-1).astype(np.float32).reshape(-1)


_TABS = np.concatenate([_pack_table(0), _pack_table(1)])
_SCALE = _make_scaling()


def _sc_interp_body(grid_ref, tabs_ref, base_ref, r0_ref, r1_ref, phr_ref, phi_ref,
                    out_ref, tv, idxb, dst, bi, r0v, r1v, phrv, phiv, outb, sem):
    wid = lax.axis_index("s") * 2 + lax.axis_index("c")
    k0 = wid * KW

    pltpu.sync_copy(tabs_ref, tv)
    pltpu.sync_copy(base_ref.at[pl.ds(k0, KW)], bi)
    pltpu.sync_copy(r0_ref.at[pl.ds(k0, KW)], r0v)
    pltpu.sync_copy(r1_ref.at[pl.ds(k0, KW)], r1v)
    pltpu.sync_copy(phr_ref.at[pl.ds(k0, KW)], phrv)
    pltpu.sync_copy(phi_ref.at[pl.ds(k0, KW)], phiv)

    iota = lax.iota(jnp.int32, 16)

    def chunk_body(ch, carry):
        cb = ch * CH

        def idx_body(g, carry2):
            bvec = bi[pl.ds(cb + g * 16, 16)]
            for j0 in range(NUMPOINTS[0]):
                for j1 in range(NUMPOINTS[1]):
                    tap = j0 * NUMPOINTS[1] + j1
                    pos = g * TAPS + tap
                    row = pos // 8
                    col = (pos % 8) * 16
                    idxb[row, pl.ds(col, 16)] = bvec + (j0 * PG + j1)
            return carry2

        lax.fori_loop(0, CH // 16, idx_body, 0)

        copies = []
        for s in range(NSEG):
            copies.append(pltpu.async_copy(
                grid_ref.at[idxb.at[s]], dst.at[pl.ds(s * 128, 128)], sem))
        for c in copies:
            c.wait()

        def grp_body(g, carry2):
            r0vec = r0v[pl.ds(cb + g * 16, 16)] * 2
            r1vec = r1v[pl.ds(cb + g * 16, 16)] * 2
            c0re = [plsc.load_gather(tv, [r0vec + (j * 2050)]) for j in range(6)]
            c0im = [plsc.load_gather(tv, [r0vec + (j * 2050 + 1)]) for j in range(6)]
            c1re = [plsc.load_gather(tv, [r1vec + (TLEN + j * 2050)]) for j in range(6)]
            c1im = [plsc.load_gather(tv, [r1vec + (TLEN + j * 2050 + 1)]) for j in range(6)]
            phr = phrv[pl.ds(cb + g * 16, 16)]
            phi = phiv[pl.ds(cb + g * 16, 16)]
            for c in range(NCOIL):
                colre = jnp.full((16,), c * 2, jnp.int32)
                colim = jnp.full((16,), c * 2 + 1, jnp.int32)
                accr = jnp.zeros((16,), jnp.float32)
                acci = jnp.zeros((16,), jnp.float32)
                for j0 in range(NUMPOINTS[0]):
                    sre = jnp.zeros((16,), jnp.float32)
                    sim = jnp.zeros((16,), jnp.float32)
                    for j1 in range(NUMPOINTS[1]):
                        tap = j0 * NUMPOINTS[1] + j1
                        rows = iota + (g * TAPS + tap) * 16
                        vre = plsc.load_gather(dst, [rows, colre])
                        vim = plsc.load_gather(dst, [rows, colim])
                        sre = sre + c1re[j1] * vre - c1im[j1] * vim
                        sim = sim + c1re[j1] * vim + c1im[j1] * vre
                    accr = accr + c0re[j0] * sre - c0im[j0] * sim
                    acci = acci + c0re[j0] * sim + c0im[j0] * sre
                outr = phr * accr - phi * acci
                outi = phr * acci + phi * accr
                outb[c, 0, pl.ds(g * 16, 16)] = outr
                outb[c, 1, pl.ds(g * 16, 16)] = outi
            return carry2

        lax.fori_loop(0, CH // 16, grp_body, 0)

        pltpu.sync_copy(outb, out_ref.at[wid, ch])
        return carry

    lax.fori_loop(0, NCH, chunk_body, 0)


@jax.jit
def _run(x, om):
    xc = (x[:, :, 0] + 1j * x[:, :, 1]) * jnp.asarray(_SCALE)
    xc = jnp.pad(xc, ((0, 0), (0, 0),
                      (0, GRID_SIZE[0] - IM_SIZE[0]),
                      (0, GRID_SIZE[1] - IM_SIZE[1])))
    Xk = jnp.fft.fftn(xc, axes=(-2, -1))[0]
    Xp = jnp.pad(Xk, ((0, 0), (0, PAD), (0, PAD)), mode="wrap")
    Xt = jnp.transpose(Xp, (1, 2, 0))
    grid = jnp.stack([jnp.real(Xt), jnp.imag(Xt)], axis=-1)
    grid = grid.reshape(NROWS, 2 * NCOIL).astype(jnp.float32)

    gsz = jnp.asarray(GRID_SIZE, dtype=om.dtype)
    tm = om * (gsz[None, :, None] / (2.0 * jnp.pi))
    tm0, tm1 = tm[0, 0], tm[0, 1]
    kf0 = jnp.floor(tm0)
    kf1 = jnp.floor(tm1)
    koff0 = kf0.astype(jnp.int32) - 2
    koff1 = kf1.astype(jnp.int32) - 2
    rb = jnp.remainder(koff0, GRID_SIZE[0])
    cbase = jnp.remainder(koff1, GRID_SIZE[1])
    base = rb * PG + cbase
    r0 = jnp.round((tm0 - koff0.astype(tm0.dtype)) * TABLE_OVERSAMP[0]).astype(jnp.int32) - 2 * TABLE_OVERSAMP[0]
    r1 = jnp.round((tm1 - koff1.astype(tm1.dtype)) * TABLE_OVERSAMP[1]).astype(jnp.int32) - 2 * TABLE_OVERSAMP[1]
    theta = om[0, 0] * jnp.float32(N_SHIFT[0]) + om[0, 1] * jnp.float32(N_SHIFT[1])
    phr = jnp.cos(theta)
    phi = jnp.sin(theta)

    tabs = jnp.asarray(_TABS)

    mesh = plsc.VectorSubcoreMesh(core_axis_name="c", subcore_axis_name="s")
    out = pl.kernel(
        _sc_interp_body,
        out_type=jax.ShapeDtypeStruct((NW, NCH, NCOIL, 2, CH), jnp.float32),
        mesh=mesh,
        compiler_params=pltpu.CompilerParams(
            needs_layout_passes=False, use_tc_tiling_on_sc=False),
        scratch_types=[
            pltpu.VMEM((2 * TLEN,), jnp.float32),
            pltpu.VMEM((NSEG, 128), jnp.int32),
            pltpu.VMEM((TAPS * CH, 2 * NCOIL), jnp.float32),
            pltpu.VMEM((KW,), jnp.int32),
            pltpu.VMEM((KW,), jnp.int32),
            pltpu.VMEM((KW,), jnp.int32),
            pltpu.VMEM((KW,), jnp.float32),
            pltpu.VMEM((KW,), jnp.float32),
            pltpu.VMEM((NCOIL, 2, CH), jnp.float32),
            pltpu.SemaphoreType.DMA,
        ],
    )(grid, tabs, base, r0, r1, phr, phi)

    kdat = jnp.transpose(out, (2, 3, 0, 1, 4)).reshape(1, NCOIL, 2, KLEN)
    return kdat


def kernel(x, om):
    return _run(x, om)
